# baseline (speedup 1.0000x reference)
import functools

import jax
import jax.numpy as jnp
from jax import lax
from jax.experimental import pallas as pl
from jax.experimental.pallas import tpu as pltpu
from jax.experimental.pallas import tpu_sc as plsc

_NC = 2
_NS = 16
_NW = _NC * _NS


@functools.lru_cache(maxsize=None)
def _make_sc_gather(n_idx, D):
    assert n_idx % (8 * _NW) == 0
    per_w = n_idx // _NW
    mesh = plsc.VectorSubcoreMesh(
        core_axis_name="c", subcore_axis_name="s",
        num_cores=_NC, num_subcores=_NS)

    @functools.partial(
        pl.kernel,
        out_type=jax.ShapeDtypeStruct((n_idx, D), jnp.float32),
        mesh=mesh,
        compiler_params=pltpu.CompilerParams(use_tc_tiling_on_sc=False),
        scratch_types=[
            pltpu.VMEM((per_w,), jnp.int32),
            pltpu.VMEM((per_w, D), jnp.float32),
            pltpu.SemaphoreType.DMA,
        ],
    )
    def gather_k(idx_hbm, table_hbm, out_hbm, idx_v, rows_v, sem):
        wid = lax.axis_index("s") * _NC + lax.axis_index("c")
        base = wid * per_w
        pltpu.sync_copy(idx_hbm.at[pl.ds(base, per_w)], idx_v)
        pltpu.async_copy(table_hbm.at[idx_v], rows_v, sem).wait()
        pltpu.sync_copy(rows_v, out_hbm.at[pl.ds(base, per_w)])

    return gather_k


def _mm_body(flatT_ref, w_ref, b_ref, o_ref):
    o_ref[...] = lax.dot_general(
        flatT_ref[...], w_ref[...],
        dimension_numbers=(((0,), (1,)), ((), ())),
        preferred_element_type=jnp.float32,
    ) + b_ref[...]


def kernel(x, table, W, b):
    B, L = x.shape
    V, D = table.shape
    K = L * D

    idx = x.reshape(-1).astype(jnp.int32)
    flatT = _make_sc_gather(B * L, D)(idx, table).reshape(B, K).T

    NBLK = 1024
    grid = (V + NBLK - 1) // NBLK
    out = pl.pallas_call(
        _mm_body,
        grid=(grid,),
        in_specs=[
            pl.BlockSpec((K, B), lambda j: (0, 0)),
            pl.BlockSpec((NBLK, K), lambda j: (0, 0)),
            pl.BlockSpec((1, NBLK), lambda j: (0, j)),
        ],
        out_specs=pl.BlockSpec((B, NBLK), lambda j: (0, 0)),
        out_shape=jax.ShapeDtypeStruct((B, V), jnp.float32),
    )(flatT, W, b.reshape(1, V))
    return out


# speedup vs baseline: 1.7724x; 1.7724x over previous
import functools

import jax
import jax.numpy as jnp
from jax import lax
from jax.experimental import pallas as pl
from jax.experimental.pallas import tpu as pltpu
from jax.experimental.pallas import tpu_sc as plsc

_NC = 2
_NS = 16
_NW = _NC * _NS


@functools.lru_cache(maxsize=None)
def _make_sc_gather(n_idx, D):
    assert n_idx % (8 * _NW) == 0
    per_w = n_idx // _NW
    mesh = plsc.VectorSubcoreMesh(
        core_axis_name="c", subcore_axis_name="s",
        num_cores=_NC, num_subcores=_NS)

    @functools.partial(
        pl.kernel,
        out_type=jax.ShapeDtypeStruct((n_idx, D), jnp.float32),
        mesh=mesh,
        compiler_params=pltpu.CompilerParams(use_tc_tiling_on_sc=False),
        scratch_types=[
            pltpu.VMEM((per_w,), jnp.int32),
            pltpu.VMEM((per_w, D), jnp.float32),
            pltpu.SemaphoreType.DMA,
        ],
    )
    def gather_k(idx_hbm, table_hbm, out_hbm, idx_v, rows_v, sem):
        wid = lax.axis_index("s") * _NC + lax.axis_index("c")
        base = wid * per_w
        pltpu.sync_copy(idx_hbm.at[pl.ds(base, per_w)], idx_v)
        pltpu.async_copy(table_hbm.at[idx_v], rows_v, sem).wait()
        pltpu.sync_copy(rows_v, out_hbm.at[pl.ds(base, per_w)])

    return gather_k


def _mm_body(w_ref, flatT_ref, b_ref, o_ref):
    o_ref[...] = lax.dot_general(
        w_ref[...], flatT_ref[...],
        dimension_numbers=(((1,), (0,)), ((), ())),
        preferred_element_type=jnp.float32,
    ) + b_ref[...]


def kernel(x, table, W, b):
    B, L = x.shape
    V, D = table.shape
    K = L * D

    idx = x.reshape(-1).astype(jnp.int32)
    flatT = _make_sc_gather(B * L, D)(idx, table).reshape(B, K).T

    NBLK = 2048
    grid = (V + NBLK - 1) // NBLK
    outT = pl.pallas_call(
        _mm_body,
        grid=(grid,),
        in_specs=[
            pl.BlockSpec((NBLK, K), lambda j: (j, 0)),
            pl.BlockSpec((K, B), lambda j: (0, 0)),
            pl.BlockSpec((NBLK, 1), lambda j: (j, 0)),
        ],
        out_specs=pl.BlockSpec((NBLK, B), lambda j: (j, 0)),
        out_shape=jax.ShapeDtypeStruct((V, B), jnp.float32),
    )(W, flatT, b.reshape(V, 1))
    return outT.T


# speedup vs baseline: 2.0162x; 1.1375x over previous
import functools

import jax
import jax.numpy as jnp
from jax import lax
from jax.experimental import pallas as pl
from jax.experimental.pallas import tpu as pltpu
from jax.experimental.pallas import tpu_sc as plsc

_NC = 2
_NS = 16
_NW = _NC * _NS


@functools.lru_cache(maxsize=None)
def _make_sc_gather(n_idx, D):
    assert n_idx % (8 * _NW) == 0
    per_w = n_idx // _NW
    mesh = plsc.VectorSubcoreMesh(
        core_axis_name="c", subcore_axis_name="s",
        num_cores=_NC, num_subcores=_NS)

    @functools.partial(
        pl.kernel,
        out_type=jax.ShapeDtypeStruct((n_idx, D), jnp.float32),
        mesh=mesh,
        compiler_params=pltpu.CompilerParams(use_tc_tiling_on_sc=False),
        scratch_types=[
            pltpu.VMEM((per_w,), jnp.int32),
            pltpu.VMEM((per_w, D), jnp.float32),
            pltpu.SemaphoreType.DMA,
        ],
    )
    def gather_k(idx_hbm, table_hbm, out_hbm, idx_v, rows_v, sem):
        wid = lax.axis_index("s") * _NC + lax.axis_index("c")
        base = wid * per_w
        pltpu.sync_copy(idx_hbm.at[pl.ds(base, per_w)], idx_v)
        pltpu.async_copy(table_hbm.at[idx_v], rows_v, sem).wait()
        pltpu.sync_copy(rows_v, out_hbm.at[pl.ds(base, per_w)])

    return gather_k


def _mm_body(w_ref, flatT_ref, b_ref, o_ref):
    acc = lax.dot_general(
        w_ref[...].astype(jnp.bfloat16), flatT_ref[...],
        dimension_numbers=(((1,), (0,)), ((), ())),
        preferred_element_type=jnp.float32,
    )
    o_ref[...] = acc + jnp.swapaxes(b_ref[...], 0, 1)


def kernel(x, table, W, b):
    B, L = x.shape
    V, D = table.shape
    K = L * D

    idx = x.reshape(-1).astype(jnp.int32)
    flatT = _make_sc_gather(B * L, D)(idx, table).reshape(B, K).T.astype(jnp.bfloat16)

    NBLK = 2048
    grid = (V + NBLK - 1) // NBLK
    outT = pl.pallas_call(
        _mm_body,
        grid=(grid,),
        in_specs=[
            pl.BlockSpec((NBLK, K), lambda j: (j, 0)),
            pl.BlockSpec((K, B), lambda j: (0, 0)),
            pl.BlockSpec((1, NBLK), lambda j: (0, j)),
        ],
        out_specs=pl.BlockSpec((NBLK, B), lambda j: (j, 0)),
        out_shape=jax.ShapeDtypeStruct((V, B), jnp.float32),
    )(W, flatT, b.reshape(1, V))
    return outT.T
